# Initial kernel scaffold; baseline (speedup 1.0000x reference)
import jax
import jax.numpy as jnp
from jax.experimental import pallas as pl


def kernel(feat, edge_index, W_fc, attn_l, attn_r, type_attn_l, type_attn_r):
    raise NotImplementedError("write your pallas kernel here")


# speedup vs baseline: 64.9146x; 64.9146x over previous
import functools
import jax
import jax.numpy as jnp
from jax import lax
from jax.experimental import pallas as pl
from jax.experimental.pallas import tpu as pltpu
from jax.experimental.pallas import tpu_sc as plsc

NC = 2
NS = 16
NW = NC * NS
CHUNK = 128


def _cdiv(a, b):
    return (a + b - 1) // b


def _tca_body(feat_ref, wt_ref, b_ref, fs_ref, sc_ref):
    x = feat_ref[...]
    fs = lax.dot_general(x, wt_ref[...], (((1,), (0,)), ((), ())),
                         precision=lax.Precision.HIGHEST,
                         preferred_element_type=jnp.float32)
    fs_ref[...] = fs
    sc_ref[...] = lax.dot_general(fs, b_ref[...], (((1,), (0,)), ((), ())),
                                  precision=lax.Precision.HIGHEST,
                                  preferred_element_type=jnp.float32)


def _tcb_body(s_ref, acc_ref, eltab_ref, edtab_ref):
    s = s_ref[...]
    sums = acc_ref[0] + acc_ref[1]
    ert = sums[:, 0:4]
    deg = sums[:, 4:5]
    degs = jnp.where(deg > 0, deg, 1.0)
    ta = jax.nn.sigmoid(s[:, 8:12] + ert / degs)
    z = jnp.zeros((s.shape[0], 8), jnp.float32)
    eltab_ref[...] = jnp.concatenate([s[:, 0:4], z, s[:, 0:4]], axis=1)
    edtab_ref[...] = jnp.concatenate([s[:, 4:8], ta, z], axis=1)


def _tcc_body(ft_ref, m_ref, out_ref):
    acc = ft_ref[0] + ft_ref[1]
    f = acc[:, 0:128]
    den = acc[:, 128:132]
    m = m_ref[...]
    recip = 1.0 / jnp.maximum(den, 1e-16)
    rb = lax.dot_general(recip, m, (((1,), (0,)), ((), ())),
                         precision=lax.Precision.HIGHEST,
                         preferred_element_type=jnp.float32)
    rst = f * rb
    n2 = lax.dot_general(rst * rst, m, (((1,), (1,)), ((), ())),
                         precision=lax.Precision.HIGHEST,
                         preferred_element_type=jnp.float32)
    inv = 1.0 / jnp.maximum(jnp.sqrt(n2), 1e-12)
    ib = lax.dot_general(inv, m, (((1,), (0,)), ((), ())),
                         precision=lax.Precision.HIGHEST,
                         preferred_element_type=jnp.float32)
    out_ref[...] = rst * ib


def _sc1_body(nchw, npt, src_hbm, dst_hbm, t1_hbm, out_hbm,
              sbuf, dbuf, g1, zbuf, acc, sem):
    cid = lax.axis_index("c")
    sid = lax.axis_index("s")
    wid = cid * NS + sid

    zeros16 = jnp.zeros((16,), jnp.float32)

    def zloop(i, _):
        zbuf[i, pl.ds(0, 16)] = zeros16
        return 0
    lax.fori_loop(0, CHUNK, zloop, 0)
    for b in range(npt // CHUNK):
        pltpu.sync_copy(zbuf, acc.at[pl.ds(sid * npt + b * CHUNK, CHUNK)])
    plsc.subcore_barrier()

    def body(k, _):
        base = wid * (nchw * CHUNK) + k * CHUNK
        pltpu.sync_copy(src_hbm.at[pl.ds(base, CHUNK)], sbuf)
        pltpu.sync_copy(dst_hbm.at[pl.ds(base, CHUNK)], dbuf)
        pltpu.async_copy(t1_hbm.at[sbuf], g1, sem).wait()
        pltpu.sync_copy(g1, acc.at[dbuf], add=True)
        return 0
    lax.fori_loop(0, nchw, body, 0)

    plsc.subcore_barrier()
    for b in range(npt // CHUNK):
        r0 = sid * npt + b * CHUNK
        pltpu.sync_copy(acc.at[pl.ds(r0, CHUNK)], out_hbm.at[cid, pl.ds(r0, CHUNK)])


def _sc2_body(nchw, npt, src_hbm, dst_hbm, el_hbm, ed_hbm, fs_hbm, out_hbm,
              sbuf, dbuf, gel, ged, rows, fused, acc, sem0, sem1, sem2):
    cid = lax.axis_index("c")
    sid = lax.axis_index("s")
    wid = cid * NS + sid
    nfull = npt // CHUNK
    rem = npt - nfull * CHUNK

    zeros16 = jnp.zeros((16,), jnp.float32)

    def zloop(i, _):
        for j in range(9):
            fused[i, pl.ds(j * 16, 16)] = zeros16
        return 0
    lax.fori_loop(0, CHUNK, zloop, 0)
    for b in range(nfull):
        pltpu.sync_copy(fused, acc.at[pl.ds(sid * npt + b * CHUNK, CHUNK)])
    if rem:
        pltpu.sync_copy(fused.at[pl.ds(0, rem)],
                        acc.at[pl.ds(sid * npt + nfull * CHUNK, rem)])
    plsc.subcore_barrier()

    iota = lax.iota(jnp.int32, 16)

    def body(k, _):
        base = wid * (nchw * CHUNK) + k * CHUNK
        pltpu.sync_copy(src_hbm.at[pl.ds(base, CHUNK)], sbuf)
        pltpu.sync_copy(dst_hbm.at[pl.ds(base, CHUNK)], dbuf)
        c1 = pltpu.async_copy(fs_hbm.at[sbuf], rows, sem0)
        c2 = pltpu.async_copy(el_hbm.at[sbuf], gel, sem1)
        c3 = pltpu.async_copy(ed_hbm.at[dbuf], ged, sem2)
        c2.wait()
        c3.wait()
        for g in range(CHUNK // 16):
            e_ids = iota + (g * 16)
            for h in range(4):
                hv = jnp.full((16,), h, jnp.int32)
                elv = plsc.load_gather(gel, [e_ids, hv])
                erv = plsc.load_gather(ged, [e_ids, hv])
                tav = plsc.load_gather(ged, [e_ids, hv + 4])
                x = (elv + erv) * tav
                act = jnp.where(x >= 0, x, 0.2 * x)
                w = jnp.exp(act)
                plsc.store_scatter(fused, [e_ids, hv + 128], w)
        c1.wait()

        def sbody(e, _):
            wv = fused[e, pl.ds(128, 16)]
            for h in range(4):
                ws = wv[h]
                for j2 in range(2):
                    j = h * 2 + j2
                    fused[e, pl.ds(j * 16, 16)] = rows[e, pl.ds(j * 16, 16)] * ws
            return 0
        lax.fori_loop(0, CHUNK, sbody, 0)

        pltpu.sync_copy(fused, acc.at[dbuf], add=True)
        return 0
    lax.fori_loop(0, nchw, body, 0)

    plsc.subcore_barrier()
    for b in range(nfull):
        r0 = sid * npt + b * CHUNK
        pltpu.sync_copy(acc.at[pl.ds(r0, CHUNK)], out_hbm.at[cid, pl.ds(r0, CHUNK)])
    if rem:
        r0 = sid * npt + nfull * CHUNK
        pltpu.sync_copy(acc.at[pl.ds(r0, rem)], out_hbm.at[cid, pl.ds(r0, rem)])


def kernel(feat, edge_index, W_fc, attn_l, attn_r, type_attn_l, type_attn_r):
    N, D = feat.shape
    H = attn_l.shape[1]
    U = attn_l.shape[2]
    HU = H * U
    E = edge_index.shape[1]

    src = edge_index[0].astype(jnp.int32)
    dst = edge_index[1].astype(jnp.int32)

    e_pad = _cdiv(E, NW * CHUNK) * NW * CHUNK
    nchw = e_pad // (NW * CHUNK)
    pad = e_pad - E
    if pad:
        src = jnp.concatenate([src, jnp.zeros((pad,), jnp.int32)])
        dst = jnp.concatenate([dst, jnp.full((pad,), N, jnp.int32)])

    NP = _cdiv(N + 1, NS * CHUNK) * NS * CHUNK
    npt = NP // NS

    Wt = W_fc.T
    mask = (jnp.arange(HU)[:, None] // U == jnp.arange(H)[None, :]).astype(jnp.float32)

    def colblk(a):
        return mask * a.reshape(HU)[:, None]
    B = jnp.concatenate([colblk(attn_l), colblk(attn_r),
                         colblk(type_attn_l), colblk(type_attn_r)], axis=1)

    BN = 1000
    fs, sc = pl.pallas_call(
        _tca_body,
        grid=(N // BN,),
        in_specs=[pl.BlockSpec((BN, D), lambda i: (i, 0)),
                  pl.BlockSpec((D, HU), lambda i: (0, 0)),
                  pl.BlockSpec((HU, 16), lambda i: (0, 0))],
        out_specs=[pl.BlockSpec((BN, HU), lambda i: (i, 0)),
                   pl.BlockSpec((BN, 16), lambda i: (i, 0))],
        out_shape=[jax.ShapeDtypeStruct((N, HU), jnp.float32),
                   jax.ShapeDtypeStruct((N, 16), jnp.float32)],
    )(feat, Wt, B)

    t1 = jnp.concatenate(
        [sc[:, 12:16], jnp.ones((N, 1), jnp.float32),
         jnp.zeros((N, 11), jnp.float32)], axis=1)

    mesh = plsc.VectorSubcoreMesh(core_axis_name="c", subcore_axis_name="s",
                                  num_cores=NC, num_subcores=NS)
    sc1 = pl.kernel(
        functools.partial(_sc1_body, nchw, npt),
        out_type=jax.ShapeDtypeStruct((NC, NP, 16), jnp.float32),
        mesh=mesh,
        compiler_params=pltpu.CompilerParams(use_tc_tiling_on_sc=False,
                                             needs_layout_passes=False),
        scratch_types=[
            pltpu.VMEM((CHUNK,), jnp.int32),
            pltpu.VMEM((CHUNK,), jnp.int32),
            pltpu.VMEM((CHUNK, 16), jnp.float32),
            pltpu.VMEM((CHUNK, 16), jnp.float32),
            pltpu.VMEM_SHARED((NP, 16), jnp.float32),
            pltpu.SemaphoreType.DMA,
        ],
    )
    acc1 = sc1(src, dst, t1)

    BP = NP // 10 if NP % 10 == 0 else NP
    s_pad = jnp.pad(sc, ((0, NP - N), (0, 0)))
    el_tab, ed_tab = pl.pallas_call(
        _tcb_body,
        grid=(NP // BP,),
        in_specs=[pl.BlockSpec((BP, 16), lambda i: (i, 0)),
                  pl.BlockSpec((NC, BP, 16), lambda i: (0, i, 0))],
        out_specs=[pl.BlockSpec((BP, 16), lambda i: (i, 0)),
                   pl.BlockSpec((BP, 16), lambda i: (i, 0))],
        out_shape=[jax.ShapeDtypeStruct((NP, 16), jnp.float32),
                   jax.ShapeDtypeStruct((NP, 16), jnp.float32)],
    )(s_pad, acc1)

    ACC_R = _cdiv(N + 1, NS * 8) * NS * 8
    npt2 = ACC_R // NS
    sc2 = pl.kernel(
        functools.partial(_sc2_body, nchw, npt2),
        out_type=jax.ShapeDtypeStruct((NC, ACC_R, 144), jnp.float32),
        mesh=mesh,
        compiler_params=pltpu.CompilerParams(use_tc_tiling_on_sc=False,
                                             needs_layout_passes=False),
        scratch_types=[
            pltpu.VMEM((CHUNK,), jnp.int32),
            pltpu.VMEM((CHUNK,), jnp.int32),
            pltpu.VMEM((CHUNK, 16), jnp.float32),
            pltpu.VMEM((CHUNK, 16), jnp.float32),
            pltpu.VMEM((CHUNK, 128), jnp.float32),
            pltpu.VMEM((CHUNK, 144), jnp.float32),
            pltpu.VMEM_SHARED((ACC_R, 144), jnp.float32),
            pltpu.SemaphoreType.DMA,
            pltpu.SemaphoreType.DMA,
            pltpu.SemaphoreType.DMA,
        ],
    )
    ft_parts = sc2(src, dst, el_tab, ed_tab, fs)

    m4 = (jnp.arange(HU)[None, :] // U == jnp.arange(H)[:, None]).astype(jnp.float32)
    out = pl.pallas_call(
        _tcc_body,
        grid=(N // BN,),
        in_specs=[pl.BlockSpec((NC, BN, 144), lambda i: (0, i, 0)),
                  pl.BlockSpec((H, HU), lambda i: (0, 0))],
        out_specs=pl.BlockSpec((BN, HU), lambda i: (i, 0)),
        out_shape=jax.ShapeDtypeStruct((N, HU), jnp.float32),
    )(ft_parts, m4)

    return out.reshape(N, H, U)


# speedup vs baseline: 122.3693x; 1.8851x over previous
import functools
import jax
import jax.numpy as jnp
from jax import lax
from jax.experimental import pallas as pl
from jax.experimental.pallas import tpu as pltpu
from jax.experimental.pallas import tpu_sc as plsc

NC = 2
NS = 16
NW = NC * NS
CH = 112


def _cdiv(a, b):
    return (a + b - 1) // b


def _tca_body(feat_ref, wt_ref, b_ref, fs_ref, sc_ref):
    x = feat_ref[...]
    fs = lax.dot_general(x, wt_ref[...], (((1,), (0,)), ((), ())),
                         precision=lax.Precision.HIGHEST,
                         preferred_element_type=jnp.float32)
    fs_ref[...] = fs
    sc_ref[...] = lax.dot_general(fs, b_ref[...], (((1,), (0,)), ((), ())),
                                  precision=lax.Precision.HIGHEST,
                                  preferred_element_type=jnp.float32)


def _tcb_body(s_ref, acc_ref, eltab_ref, edtab_ref):
    s = s_ref[...]
    sums = acc_ref[0] + acc_ref[1]
    ert = sums[:, 0:4]
    deg = sums[:, 4:5]
    degs = jnp.where(deg > 0, deg, 1.0)
    ta = jax.nn.sigmoid(s[:, 8:12] + ert / degs)
    z = jnp.zeros((s.shape[0], 8), jnp.float32)
    eltab_ref[...] = jnp.concatenate([s[:, 0:4], z, s[:, 0:4]], axis=1)
    edtab_ref[...] = jnp.concatenate([s[:, 4:8], ta, z], axis=1)


def _tcc_body(ft_ref, dn_ref, m_ref, out_ref):
    f = ft_ref[0] + ft_ref[1]
    dsum = dn_ref[0] + dn_ref[1]
    den = dsum[:, 8:12]
    m = m_ref[...]
    recip = 1.0 / jnp.maximum(den, 1e-16)
    rb = lax.dot_general(recip, m, (((1,), (0,)), ((), ())),
                         precision=lax.Precision.HIGHEST,
                         preferred_element_type=jnp.float32)
    rst = f * rb
    n2 = lax.dot_general(rst * rst, m, (((1,), (1,)), ((), ())),
                         precision=lax.Precision.HIGHEST,
                         preferred_element_type=jnp.float32)
    inv = 1.0 / jnp.maximum(jnp.sqrt(n2), 1e-12)
    ib = lax.dot_general(inv, m, (((1,), (0,)), ((), ())),
                         precision=lax.Precision.HIGHEST,
                         preferred_element_type=jnp.float32)
    out_ref[...] = rst * ib


def _sc1_body(nchw, npt, src_hbm, dst_hbm, t1_hbm, out_hbm,
              sb0, db0, sb1, db1, g10, g11, zbuf, acc, si, sj, sg):
    cid = lax.axis_index("c")
    sid = lax.axis_index("s")
    wid = cid * NS + sid
    SB = [sb0, sb1]
    DB = [db0, db1]
    G1 = [g10, g11]
    nfull = npt // CH
    rem = npt - nfull * CH
    zeros16 = jnp.zeros((16,), jnp.float32)

    def zloop(i, _):
        zbuf[i, pl.ds(0, 16)] = zeros16
        return 0
    lax.fori_loop(0, CH, zloop, 0)
    for b in range(nfull):
        pltpu.sync_copy(zbuf, acc.at[pl.ds(sid * npt + b * CH, CH)])
    if rem:
        pltpu.sync_copy(zbuf.at[pl.ds(0, rem)],
                        acc.at[pl.ds(sid * npt + nfull * CH, rem)])
    plsc.subcore_barrier()

    ebase = wid * (nchw * CH)

    def idx_issue(k, s):
        pltpu.async_copy(src_hbm.at[pl.ds(ebase + k * CH, CH)], SB[s], si)
        pltpu.async_copy(dst_hbm.at[pl.ds(ebase + k * CH, CH)], DB[s], sj)

    def idx_wait(k, s):
        pltpu.make_async_copy(src_hbm.at[pl.ds(ebase + k * CH, CH)], SB[s], si).wait()
        pltpu.make_async_copy(dst_hbm.at[pl.ds(ebase + k * CH, CH)], DB[s], sj).wait()

    def gather_issue(s):
        pltpu.async_copy(t1_hbm.at[SB[s]], G1[s], sg)

    def gather_wait(s):
        pltpu.make_async_copy(t1_hbm.at[SB[s]], G1[s], sg).wait()

    def scat(s):
        pltpu.sync_copy(G1[s], acc.at[DB[s]], add=True)

    idx_issue(0, 0)
    idx_wait(0, 0)
    gather_issue(0)
    idx_issue(1, 1)

    def pair(k2, _):
        for par in (0, 1):
            k = 2 * k2 + par
            s, s2 = par, 1 - par
            idx_wait(k + 1, s2)
            gather_issue(s2)
            gather_wait(s)
            scat(s)
            idx_issue(k + 2, s)
        return 0
    lax.fori_loop(0, (nchw - 2) // 2, pair, 0)

    k = nchw - 2
    idx_wait(k + 1, 1)
    gather_issue(1)
    gather_wait(0)
    scat(0)
    gather_wait(1)
    scat(1)

    plsc.subcore_barrier()
    for b in range(nfull):
        r0 = sid * npt + b * CH
        pltpu.sync_copy(acc.at[pl.ds(r0, CH)], out_hbm.at[cid, pl.ds(r0, CH)])
    if rem:
        r0 = sid * npt + nfull * CH
        pltpu.sync_copy(acc.at[pl.ds(r0, rem)], out_hbm.at[cid, pl.ds(r0, rem)])


def _sc2_body(nchw, npt, src_hbm, dst_hbm, el_hbm, ed_hbm, fs_hbm,
              ft_hbm, dn_hbm,
              sb0, db0, sb1, db1, ge0, gd0, ge1, gd1, fu0, fu1,
              ft_acc, dn_acc, si, sj, se, sd, sr):
    cid = lax.axis_index("c")
    sid = lax.axis_index("s")
    wid = cid * NS + sid
    SB = [sb0, sb1]
    DB = [db0, db1]
    GE = [ge0, ge1]
    GD = [gd0, gd1]
    FU = [fu0, fu1]
    nfull = npt // CH
    rem = npt - nfull * CH
    zeros16 = jnp.zeros((16,), jnp.float32)
    iota = lax.iota(jnp.int32, 16)

    def zloop(i, _):
        for j in range(8):
            fu0[i, pl.ds(j * 16, 16)] = zeros16
        gd0[i, pl.ds(0, 16)] = zeros16
        return 0
    lax.fori_loop(0, CH, zloop, 0)
    for b in range(nfull):
        r0 = sid * npt + b * CH
        pltpu.sync_copy(fu0, ft_acc.at[pl.ds(r0, CH)])
        pltpu.sync_copy(gd0, dn_acc.at[pl.ds(r0, CH)])
    if rem:
        r0 = sid * npt + nfull * CH
        pltpu.sync_copy(fu0.at[pl.ds(0, rem)], ft_acc.at[pl.ds(r0, rem)])
        pltpu.sync_copy(gd0.at[pl.ds(0, rem)], dn_acc.at[pl.ds(r0, rem)])
    plsc.subcore_barrier()

    ebase = wid * (nchw * CH)

    def idx_issue(k, s):
        pltpu.async_copy(src_hbm.at[pl.ds(ebase + k * CH, CH)], SB[s], si)
        pltpu.async_copy(dst_hbm.at[pl.ds(ebase + k * CH, CH)], DB[s], sj)

    def idx_wait(k, s):
        pltpu.make_async_copy(src_hbm.at[pl.ds(ebase + k * CH, CH)], SB[s], si).wait()
        pltpu.make_async_copy(dst_hbm.at[pl.ds(ebase + k * CH, CH)], DB[s], sj).wait()

    def gather_issue(s):
        pltpu.async_copy(fs_hbm.at[SB[s]], FU[s], sr)
        pltpu.async_copy(el_hbm.at[SB[s]], GE[s], se)
        pltpu.async_copy(ed_hbm.at[DB[s]], GD[s], sd)

    def small_wait(s):
        pltpu.make_async_copy(el_hbm.at[SB[s]], GE[s], se).wait()
        pltpu.make_async_copy(ed_hbm.at[DB[s]], GD[s], sd).wait()

    def rows_wait(s):
        pltpu.make_async_copy(fs_hbm.at[SB[s]], FU[s], sr).wait()

    def compute_w(s):
        for g in range(CH // 16):
            e_ids = iota + (g * 16)
            for h in range(4):
                hv = jnp.full((16,), h, jnp.int32)
                elv = plsc.load_gather(GE[s], [e_ids, hv])
                erv = plsc.load_gather(GD[s], [e_ids, hv])
                tav = plsc.load_gather(GD[s], [e_ids, hv + 4])
                x = (elv + erv) * tav
                act = jnp.where(x >= 0, x, 0.2 * x)
                w = jnp.exp(act)
                plsc.store_scatter(GD[s], [e_ids, hv + 8], w)

    def scale(s):
        fu = FU[s]
        gd = GD[s]

        def sbody(e, _):
            wv = gd[e, pl.ds(0, 16)]
            for h in range(4):
                ws = wv[8 + h]
                for j2 in range(2):
                    j = h * 2 + j2
                    fu[e, pl.ds(j * 16, 16)] = fu[e, pl.ds(j * 16, 16)] * ws
            return 0
        lax.fori_loop(0, CH, sbody, 0)

    def scat(s):
        pltpu.sync_copy(FU[s], ft_acc.at[DB[s]], add=True)
        pltpu.sync_copy(GD[s], dn_acc.at[DB[s]], add=True)

    idx_issue(0, 0)
    idx_wait(0, 0)
    gather_issue(0)
    idx_issue(1, 1)

    def pair(k2, _):
        for par in (0, 1):
            k = 2 * k2 + par
            s, s2 = par, 1 - par
            idx_wait(k + 1, s2)
            gather_issue(s2)
            small_wait(s)
            compute_w(s)
            rows_wait(s)
            scale(s)
            scat(s)
            idx_issue(k + 2, s)
        return 0
    lax.fori_loop(0, (nchw - 2) // 2, pair, 0)

    k = nchw - 2
    idx_wait(k + 1, 1)
    gather_issue(1)
    small_wait(0)
    compute_w(0)
    rows_wait(0)
    scale(0)
    scat(0)
    small_wait(1)
    compute_w(1)
    rows_wait(1)
    scale(1)
    scat(1)

    plsc.subcore_barrier()
    for b in range(nfull):
        r0 = sid * npt + b * CH
        pltpu.sync_copy(ft_acc.at[pl.ds(r0, CH)], ft_hbm.at[cid, pl.ds(r0, CH)])
        pltpu.sync_copy(dn_acc.at[pl.ds(r0, CH)], dn_hbm.at[cid, pl.ds(r0, CH)])
    if rem:
        r0 = sid * npt + nfull * CH
        pltpu.sync_copy(ft_acc.at[pl.ds(r0, rem)], ft_hbm.at[cid, pl.ds(r0, rem)])
        pltpu.sync_copy(dn_acc.at[pl.ds(r0, rem)], dn_hbm.at[cid, pl.ds(r0, rem)])


def kernel(feat, edge_index, W_fc, attn_l, attn_r, type_attn_l, type_attn_r):
    N, D = feat.shape
    H = attn_l.shape[1]
    U = attn_l.shape[2]
    HU = H * U
    E = edge_index.shape[1]

    src = edge_index[0].astype(jnp.int32)
    dst = edge_index[1].astype(jnp.int32)

    nchw = _cdiv(E, NW * CH)
    if nchw % 2:
        nchw += 1
    e_pad = nchw * NW * CH
    pad = e_pad - E
    if pad:
        src = jnp.concatenate([src, jnp.zeros((pad,), jnp.int32)])
        dst = jnp.concatenate([dst, jnp.full((pad,), N, jnp.int32)])

    NP = _cdiv(N + 1, NS * 128) * NS * 128
    npt = NP // NS
    ACC_R = _cdiv(N + 1, NS * 8) * NS * 8
    npt2 = ACC_R // NS

    Wt = W_fc.T
    mask = (jnp.arange(HU)[:, None] // U == jnp.arange(H)[None, :]).astype(jnp.float32)

    def colblk(a):
        return mask * a.reshape(HU)[:, None]
    B = jnp.concatenate([colblk(attn_l), colblk(attn_r),
                         colblk(type_attn_l), colblk(type_attn_r)], axis=1)

    BN = 1000
    fs, sc = pl.pallas_call(
        _tca_body,
        grid=(N // BN,),
        in_specs=[pl.BlockSpec((BN, D), lambda i: (i, 0)),
                  pl.BlockSpec((D, HU), lambda i: (0, 0)),
                  pl.BlockSpec((HU, 16), lambda i: (0, 0))],
        out_specs=[pl.BlockSpec((BN, HU), lambda i: (i, 0)),
                   pl.BlockSpec((BN, 16), lambda i: (i, 0))],
        out_shape=[jax.ShapeDtypeStruct((N, HU), jnp.float32),
                   jax.ShapeDtypeStruct((N, 16), jnp.float32)],
    )(feat, Wt, B)

    t1 = jnp.concatenate(
        [sc[:, 12:16], jnp.ones((N, 1), jnp.float32),
         jnp.zeros((N, 11), jnp.float32)], axis=1)

    mesh = plsc.VectorSubcoreMesh(core_axis_name="c", subcore_axis_name="s",
                                  num_cores=NC, num_subcores=NS)
    sc1 = pl.kernel(
        functools.partial(_sc1_body, nchw, npt),
        out_type=jax.ShapeDtypeStruct((NC, NP, 16), jnp.float32),
        mesh=mesh,
        compiler_params=pltpu.CompilerParams(use_tc_tiling_on_sc=False,
                                             needs_layout_passes=False),
        scratch_types=[
            pltpu.VMEM((CH,), jnp.int32),
            pltpu.VMEM((CH,), jnp.int32),
            pltpu.VMEM((CH,), jnp.int32),
            pltpu.VMEM((CH,), jnp.int32),
            pltpu.VMEM((CH, 16), jnp.float32),
            pltpu.VMEM((CH, 16), jnp.float32),
            pltpu.VMEM((CH, 16), jnp.float32),
            pltpu.VMEM_SHARED((NP, 16), jnp.float32),
            pltpu.SemaphoreType.DMA,
            pltpu.SemaphoreType.DMA,
            pltpu.SemaphoreType.DMA,
        ],
    )
    acc1 = sc1(src, dst, t1)

    BP = NP // 10 if NP % 10 == 0 else NP
    s_pad = jnp.pad(sc, ((0, NP - N), (0, 0)))
    el_tab, ed_tab = pl.pallas_call(
        _tcb_body,
        grid=(NP // BP,),
        in_specs=[pl.BlockSpec((BP, 16), lambda i: (i, 0)),
                  pl.BlockSpec((NC, BP, 16), lambda i: (0, i, 0))],
        out_specs=[pl.BlockSpec((BP, 16), lambda i: (i, 0)),
                   pl.BlockSpec((BP, 16), lambda i: (i, 0))],
        out_shape=[jax.ShapeDtypeStruct((NP, 16), jnp.float32),
                   jax.ShapeDtypeStruct((NP, 16), jnp.float32)],
    )(s_pad, acc1)

    sc2 = pl.kernel(
        functools.partial(_sc2_body, nchw, npt2),
        out_type=[jax.ShapeDtypeStruct((NC, ACC_R, 128), jnp.float32),
                  jax.ShapeDtypeStruct((NC, ACC_R, 16), jnp.float32)],
        mesh=mesh,
        compiler_params=pltpu.CompilerParams(use_tc_tiling_on_sc=False,
                                             needs_layout_passes=False),
        scratch_types=[
            pltpu.VMEM((CH,), jnp.int32),
            pltpu.VMEM((CH,), jnp.int32),
            pltpu.VMEM((CH,), jnp.int32),
            pltpu.VMEM((CH,), jnp.int32),
            pltpu.VMEM((CH, 16), jnp.float32),
            pltpu.VMEM((CH, 16), jnp.float32),
            pltpu.VMEM((CH, 16), jnp.float32),
            pltpu.VMEM((CH, 16), jnp.float32),
            pltpu.VMEM((CH, 128), jnp.float32),
            pltpu.VMEM((CH, 128), jnp.float32),
            pltpu.VMEM_SHARED((ACC_R, 128), jnp.float32),
            pltpu.VMEM_SHARED((ACC_R, 16), jnp.float32),
            pltpu.SemaphoreType.DMA,
            pltpu.SemaphoreType.DMA,
            pltpu.SemaphoreType.DMA,
            pltpu.SemaphoreType.DMA,
            pltpu.SemaphoreType.DMA,
        ],
    )
    ft_parts, dn_parts = sc2(src, dst, el_tab, ed_tab, fs)

    m4 = (jnp.arange(HU)[None, :] // U == jnp.arange(H)[:, None]).astype(jnp.float32)
    out = pl.pallas_call(
        _tcc_body,
        grid=(N // BN,),
        in_specs=[pl.BlockSpec((NC, BN, 128), lambda i: (0, i, 0)),
                  pl.BlockSpec((NC, BN, 16), lambda i: (0, i, 0)),
                  pl.BlockSpec((H, HU), lambda i: (0, 0))],
        out_specs=pl.BlockSpec((BN, HU), lambda i: (i, 0)),
        out_shape=jax.ShapeDtypeStruct((N, HU), jnp.float32),
    )(ft_parts, dn_parts, m4)

    return out.reshape(N, H, U)


# speedup vs baseline: 166.9795x; 1.3646x over previous
import functools
import jax
import jax.numpy as jnp
from jax import lax
from jax.experimental import pallas as pl
from jax.experimental.pallas import tpu as pltpu
from jax.experimental.pallas import tpu_sc as plsc

NC = 2
NS = 16
NW = NC * NS
CH = 112


def _cdiv(a, b):
    return (a + b - 1) // b


def _tca_body(feat_ref, wt_ref, b_ref, fs_ref, sc_ref):
    x = feat_ref[...]
    fs = lax.dot_general(x, wt_ref[...], (((1,), (0,)), ((), ())),
                         precision=lax.Precision.HIGHEST,
                         preferred_element_type=jnp.float32)
    fs_ref[...] = fs
    sc_ref[...] = lax.dot_general(fs, b_ref[...], (((1,), (0,)), ((), ())),
                                  precision=lax.Precision.HIGHEST,
                                  preferred_element_type=jnp.float32)


def _tcb_body(s_ref, acc_ref, eltab_ref, edtab_ref):
    s = s_ref[...]
    sums = acc_ref[0] + acc_ref[1]
    ert = sums[:, 0:4]
    deg = sums[:, 4:5]
    degs = jnp.where(deg > 0, deg, 1.0)
    ta = jax.nn.sigmoid(s[:, 8:12] + ert / degs)
    z = jnp.zeros((s.shape[0], 8), jnp.float32)
    eltab_ref[...] = jnp.concatenate([s[:, 0:4], z, s[:, 0:4]], axis=1)
    edtab_ref[...] = jnp.concatenate([s[:, 4:8], ta, z], axis=1)


def _tcc_body(ft_ref, dn_ref, m_ref, out_ref):
    f = ft_ref[0] + ft_ref[1]
    dsum = dn_ref[0] + dn_ref[1]
    den = dsum[:, 8:12]
    m = m_ref[...]
    recip = 1.0 / jnp.maximum(den, 1e-16)
    rb = lax.dot_general(recip, m, (((1,), (0,)), ((), ())),
                         precision=lax.Precision.HIGHEST,
                         preferred_element_type=jnp.float32)
    rst = f * rb
    n2 = lax.dot_general(rst * rst, m, (((1,), (1,)), ((), ())),
                         precision=lax.Precision.HIGHEST,
                         preferred_element_type=jnp.float32)
    inv = 1.0 / jnp.maximum(jnp.sqrt(n2), 1e-12)
    ib = lax.dot_general(inv, m, (((1,), (0,)), ((), ())),
                         precision=lax.Precision.HIGHEST,
                         preferred_element_type=jnp.float32)
    out_ref[...] = rst * ib


def _sc1_body(nchw, npt, src_hbm, dst_hbm, t1_hbm, out_hbm,
              sb0, db0, sb1, db1, ds0, ds1, g10, g11, zbuf, acc, si, sj, sg, sf):
    cid = lax.axis_index("c")
    sid = lax.axis_index("s")
    wid = cid * NS + sid
    SB = [sb0, sb1]
    DB = [db0, db1]
    DS = [ds0, ds1]
    G1 = [g10, g11]
    nfull = npt // CH
    rem = npt - nfull * CH
    zeros16 = jnp.zeros((16,), jnp.float32)
    zeros16i = jnp.zeros((16,), jnp.int32)

    def zloop(i, _):
        zbuf[i, pl.ds(0, 16)] = zeros16
        return 0
    lax.fori_loop(0, CH, zloop, 0)

    def ziloop(i, _):
        ds1[pl.ds(i * 16, 16)] = zeros16i
        return 0
    lax.fori_loop(0, CH // 16, ziloop, 0)
    for b in range(nfull):
        pltpu.sync_copy(zbuf, acc.at[pl.ds(sid * npt + b * CH, CH)])
    if rem:
        pltpu.sync_copy(zbuf.at[pl.ds(0, rem)],
                        acc.at[pl.ds(sid * npt + nfull * CH, rem)])
    plsc.subcore_barrier()

    ebase = wid * (nchw * CH)

    def idx_issue(k, s):
        pltpu.async_copy(src_hbm.at[pl.ds(ebase + k * CH, CH)], SB[s], si)
        pltpu.async_copy(dst_hbm.at[pl.ds(ebase + k * CH, CH)], DB[s], sj)

    def idx_wait(k, s):
        pltpu.make_async_copy(src_hbm.at[pl.ds(ebase + k * CH, CH)], SB[s], si).wait()
        pltpu.make_async_copy(dst_hbm.at[pl.ds(ebase + k * CH, CH)], DB[s], sj).wait()

    def gather_issue(s):
        pltpu.async_copy(t1_hbm.at[SB[s]], G1[s], sg)

    def gather_wait(s):
        pltpu.make_async_copy(t1_hbm.at[SB[s]], G1[s], sg).wait()

    def idx_snap(s):
        def cbody(i, _):
            DS[s][pl.ds(i * 16, 16)] = DB[s][pl.ds(i * 16, 16)]
            return 0
        lax.fori_loop(0, CH // 16, cbody, 0)

    def scat_issue(s):
        pltpu.async_copy(G1[s], acc.at[DS[s]], sf, add=True)

    def scat_wait(s):
        pltpu.make_async_copy(G1[s], acc.at[DS[s]], sf).wait()

    pltpu.async_copy(zbuf, acc.at[ds1], sf, add=True)

    idx_issue(0, 0)
    idx_wait(0, 0)
    gather_issue(0)
    idx_issue(1, 1)

    def pair(k2, _):
        for par in (0, 1):
            k = 2 * k2 + par
            s, s2 = par, 1 - par
            scat_wait(s2)
            idx_wait(k + 1, s2)
            gather_issue(s2)
            gather_wait(s)
            idx_snap(s)
            idx_issue(k + 2, s)
            scat_issue(s)
        return 0
    lax.fori_loop(0, (nchw - 2) // 2, pair, 0)

    k = nchw - 2
    scat_wait(1)
    idx_wait(k + 1, 1)
    gather_issue(1)
    gather_wait(0)
    idx_snap(0)
    scat_issue(0)
    scat_wait(0)
    gather_wait(1)
    idx_snap(1)
    scat_issue(1)
    scat_wait(1)

    plsc.subcore_barrier()
    for b in range(nfull):
        r0 = sid * npt + b * CH
        pltpu.sync_copy(acc.at[pl.ds(r0, CH)], out_hbm.at[cid, pl.ds(r0, CH)])
    if rem:
        r0 = sid * npt + nfull * CH
        pltpu.sync_copy(acc.at[pl.ds(r0, rem)], out_hbm.at[cid, pl.ds(r0, rem)])


def _sc2_body(nchw, npt, src_hbm, dst_hbm, el_hbm, ed_hbm, fs_hbm,
              ft_hbm, dn_hbm,
              sb0, db0, sb1, db1, ds0, ds1, ge0, gd0, ge1, gd1, fu0, fu1,
              ft_acc, dn_acc, si, sj, se, sd, sr, sf, sn):
    cid = lax.axis_index("c")
    sid = lax.axis_index("s")
    wid = cid * NS + sid
    SB = [sb0, sb1]
    DB = [db0, db1]
    DS = [ds0, ds1]
    GE = [ge0, ge1]
    GD = [gd0, gd1]
    FU = [fu0, fu1]
    nfull = npt // CH
    rem = npt - nfull * CH
    zeros16 = jnp.zeros((16,), jnp.float32)
    zeros16i = jnp.zeros((16,), jnp.int32)
    iota = lax.iota(jnp.int32, 16)

    def zloop(i, _):
        for j in range(8):
            fu0[i, pl.ds(j * 16, 16)] = zeros16
            fu1[i, pl.ds(j * 16, 16)] = zeros16
        gd0[i, pl.ds(0, 16)] = zeros16
        gd1[i, pl.ds(0, 16)] = zeros16
        return 0
    lax.fori_loop(0, CH, zloop, 0)

    def ziloop(i, _):
        ds1[pl.ds(i * 16, 16)] = zeros16i
        return 0
    lax.fori_loop(0, CH // 16, ziloop, 0)
    for b in range(nfull):
        r0 = sid * npt + b * CH
        pltpu.sync_copy(fu0, ft_acc.at[pl.ds(r0, CH)])
        pltpu.sync_copy(gd0, dn_acc.at[pl.ds(r0, CH)])
    if rem:
        r0 = sid * npt + nfull * CH
        pltpu.sync_copy(fu0.at[pl.ds(0, rem)], ft_acc.at[pl.ds(r0, rem)])
        pltpu.sync_copy(gd0.at[pl.ds(0, rem)], dn_acc.at[pl.ds(r0, rem)])
    plsc.subcore_barrier()

    ebase = wid * (nchw * CH)

    def idx_issue(k, s):
        pltpu.async_copy(src_hbm.at[pl.ds(ebase + k * CH, CH)], SB[s], si)
        pltpu.async_copy(dst_hbm.at[pl.ds(ebase + k * CH, CH)], DB[s], sj)

    def idx_wait(k, s):
        pltpu.make_async_copy(src_hbm.at[pl.ds(ebase + k * CH, CH)], SB[s], si).wait()
        pltpu.make_async_copy(dst_hbm.at[pl.ds(ebase + k * CH, CH)], DB[s], sj).wait()

    def gather_issue(s):
        pltpu.async_copy(fs_hbm.at[SB[s]], FU[s], sr)
        pltpu.async_copy(el_hbm.at[SB[s]], GE[s], se)
        pltpu.async_copy(ed_hbm.at[DB[s]], GD[s], sd)

    def small_wait(s):
        pltpu.make_async_copy(el_hbm.at[SB[s]], GE[s], se).wait()
        pltpu.make_async_copy(ed_hbm.at[DB[s]], GD[s], sd).wait()

    def rows_wait(s):
        pltpu.make_async_copy(fs_hbm.at[SB[s]], FU[s], sr).wait()

    def compute_w(s):
        for g in range(CH // 16):
            e_ids = iota + (g * 16)
            for h in range(4):
                hv = jnp.full((16,), h, jnp.int32)
                elv = plsc.load_gather(GE[s], [e_ids, hv])
                erv = plsc.load_gather(GD[s], [e_ids, hv])
                tav = plsc.load_gather(GD[s], [e_ids, hv + 4])
                x = (elv + erv) * tav
                act = jnp.where(x >= 0, x, 0.2 * x)
                w = jnp.exp(act)
                plsc.store_scatter(GD[s], [e_ids, hv + 8], w)

    def scale(s):
        fu = FU[s]
        gd = GD[s]

        def sbody(e):
            wv = gd[e, pl.ds(0, 16)]
            for h in range(4):
                ws = wv[8 + h]
                for j2 in range(2):
                    j = h * 2 + j2
                    fu[e, pl.ds(j * 16, 16)] = fu[e, pl.ds(j * 16, 16)] * ws
        plsc.parallel_loop(0, CH, 1, unroll=8)(sbody)

    def idx_snap(s):
        def cbody(i, _):
            DS[s][pl.ds(i * 16, 16)] = DB[s][pl.ds(i * 16, 16)]
            return 0
        lax.fori_loop(0, CH // 16, cbody, 0)

    def scat_issue(s):
        pltpu.async_copy(FU[s], ft_acc.at[DS[s]], sf, add=True)
        pltpu.async_copy(GD[s], dn_acc.at[DS[s]], sn, add=True)

    def scat_wait(s):
        pltpu.make_async_copy(FU[s], ft_acc.at[DS[s]], sf).wait()
        pltpu.make_async_copy(GD[s], dn_acc.at[DS[s]], sn).wait()

    scat_issue(1)

    idx_issue(0, 0)
    idx_wait(0, 0)
    gather_issue(0)
    idx_issue(1, 1)

    def pair(k2, _):
        for par in (0, 1):
            k = 2 * k2 + par
            s, s2 = par, 1 - par
            scat_wait(s2)
            idx_wait(k + 1, s2)
            gather_issue(s2)
            small_wait(s)
            compute_w(s)
            idx_snap(s)
            rows_wait(s)
            scale(s)
            idx_issue(k + 2, s)
            scat_issue(s)
        return 0
    lax.fori_loop(0, (nchw - 2) // 2, pair, 0)

    k = nchw - 2
    scat_wait(1)
    idx_wait(k + 1, 1)
    gather_issue(1)
    small_wait(0)
    compute_w(0)
    idx_snap(0)
    rows_wait(0)
    scale(0)
    scat_issue(0)
    scat_wait(0)
    small_wait(1)
    compute_w(1)
    idx_snap(1)
    rows_wait(1)
    scale(1)
    scat_issue(1)
    scat_wait(1)

    plsc.subcore_barrier()
    for b in range(nfull):
        r0 = sid * npt + b * CH
        pltpu.sync_copy(ft_acc.at[pl.ds(r0, CH)], ft_hbm.at[cid, pl.ds(r0, CH)])
        pltpu.sync_copy(dn_acc.at[pl.ds(r0, CH)], dn_hbm.at[cid, pl.ds(r0, CH)])
    if rem:
        r0 = sid * npt + nfull * CH
        pltpu.sync_copy(ft_acc.at[pl.ds(r0, rem)], ft_hbm.at[cid, pl.ds(r0, rem)])
        pltpu.sync_copy(dn_acc.at[pl.ds(r0, rem)], dn_hbm.at[cid, pl.ds(r0, rem)])


def kernel(feat, edge_index, W_fc, attn_l, attn_r, type_attn_l, type_attn_r):
    N, D = feat.shape
    H = attn_l.shape[1]
    U = attn_l.shape[2]
    HU = H * U
    E = edge_index.shape[1]

    src = edge_index[0].astype(jnp.int32)
    dst = edge_index[1].astype(jnp.int32)

    NP = _cdiv(N + 1, NS * 128) * NS * 128
    npt = NP // NS
    ACC_R = _cdiv(N + 1, NS * 8) * NS * 8
    npt2 = ACC_R // NS

    nchw = _cdiv(E, NW * CH)
    if nchw % 2:
        nchw += 1
    e_pad = nchw * NW * CH
    pad = e_pad - E
    if pad:
        ar = jnp.arange(pad, dtype=jnp.int32)
        src = jnp.concatenate([src, ar % N])
        dst = jnp.concatenate([dst, N + ar % (ACC_R - N)])

    Wt = W_fc.T
    mask = (jnp.arange(HU)[:, None] // U == jnp.arange(H)[None, :]).astype(jnp.float32)

    def colblk(a):
        return mask * a.reshape(HU)[:, None]
    B = jnp.concatenate([colblk(attn_l), colblk(attn_r),
                         colblk(type_attn_l), colblk(type_attn_r)], axis=1)

    BN = 1000
    fs, sc = pl.pallas_call(
        _tca_body,
        grid=(N // BN,),
        in_specs=[pl.BlockSpec((BN, D), lambda i: (i, 0)),
                  pl.BlockSpec((D, HU), lambda i: (0, 0)),
                  pl.BlockSpec((HU, 16), lambda i: (0, 0))],
        out_specs=[pl.BlockSpec((BN, HU), lambda i: (i, 0)),
                   pl.BlockSpec((BN, 16), lambda i: (i, 0))],
        out_shape=[jax.ShapeDtypeStruct((N, HU), jnp.float32),
                   jax.ShapeDtypeStruct((N, 16), jnp.float32)],
    )(feat, Wt, B)

    t1 = jnp.concatenate(
        [sc[:, 12:16], jnp.ones((N, 1), jnp.float32),
         jnp.zeros((N, 11), jnp.float32)], axis=1)

    mesh = plsc.VectorSubcoreMesh(core_axis_name="c", subcore_axis_name="s",
                                  num_cores=NC, num_subcores=NS)
    sc1 = pl.kernel(
        functools.partial(_sc1_body, nchw, npt),
        out_type=jax.ShapeDtypeStruct((NC, NP, 16), jnp.float32),
        mesh=mesh,
        compiler_params=pltpu.CompilerParams(use_tc_tiling_on_sc=False,
                                             needs_layout_passes=False),
        scratch_types=[
            pltpu.VMEM((CH,), jnp.int32),
            pltpu.VMEM((CH,), jnp.int32),
            pltpu.VMEM((CH,), jnp.int32),
            pltpu.VMEM((CH,), jnp.int32),
            pltpu.VMEM((CH,), jnp.int32),
            pltpu.VMEM((CH,), jnp.int32),
            pltpu.VMEM((CH, 16), jnp.float32),
            pltpu.VMEM((CH, 16), jnp.float32),
            pltpu.VMEM((CH, 16), jnp.float32),
            pltpu.VMEM_SHARED((NP, 16), jnp.float32),
            pltpu.SemaphoreType.DMA,
            pltpu.SemaphoreType.DMA,
            pltpu.SemaphoreType.DMA,
            pltpu.SemaphoreType.DMA,
        ],
    )
    acc1 = sc1(src, dst, t1)

    BP = NP // 10 if NP % 10 == 0 else NP
    s_pad = jnp.pad(sc, ((0, NP - N), (0, 0)))
    el_tab, ed_tab = pl.pallas_call(
        _tcb_body,
        grid=(NP // BP,),
        in_specs=[pl.BlockSpec((BP, 16), lambda i: (i, 0)),
                  pl.BlockSpec((NC, BP, 16), lambda i: (0, i, 0))],
        out_specs=[pl.BlockSpec((BP, 16), lambda i: (i, 0)),
                   pl.BlockSpec((BP, 16), lambda i: (i, 0))],
        out_shape=[jax.ShapeDtypeStruct((NP, 16), jnp.float32),
                   jax.ShapeDtypeStruct((NP, 16), jnp.float32)],
    )(s_pad, acc1)

    sc2 = pl.kernel(
        functools.partial(_sc2_body, nchw, npt2),
        out_type=[jax.ShapeDtypeStruct((NC, ACC_R, 128), jnp.float32),
                  jax.ShapeDtypeStruct((NC, ACC_R, 16), jnp.float32)],
        mesh=mesh,
        compiler_params=pltpu.CompilerParams(use_tc_tiling_on_sc=False,
                                             needs_layout_passes=False),
        scratch_types=[
            pltpu.VMEM((CH,), jnp.int32),
            pltpu.VMEM((CH,), jnp.int32),
            pltpu.VMEM((CH,), jnp.int32),
            pltpu.VMEM((CH,), jnp.int32),
            pltpu.VMEM((CH,), jnp.int32),
            pltpu.VMEM((CH,), jnp.int32),
            pltpu.VMEM((CH, 16), jnp.float32),
            pltpu.VMEM((CH, 16), jnp.float32),
            pltpu.VMEM((CH, 16), jnp.float32),
            pltpu.VMEM((CH, 16), jnp.float32),
            pltpu.VMEM((CH, 128), jnp.float32),
            pltpu.VMEM((CH, 128), jnp.float32),
            pltpu.VMEM_SHARED((ACC_R, 128), jnp.float32),
            pltpu.VMEM_SHARED((ACC_R, 16), jnp.float32),
            pltpu.SemaphoreType.DMA,
            pltpu.SemaphoreType.DMA,
            pltpu.SemaphoreType.DMA,
            pltpu.SemaphoreType.DMA,
            pltpu.SemaphoreType.DMA,
            pltpu.SemaphoreType.DMA,
            pltpu.SemaphoreType.DMA,
        ],
    )
    ft_parts, dn_parts = sc2(src, dst, el_tab, ed_tab, fs)

    m4 = (jnp.arange(HU)[None, :] // U == jnp.arange(H)[:, None]).astype(jnp.float32)
    out = pl.pallas_call(
        _tcc_body,
        grid=(N // BN,),
        in_specs=[pl.BlockSpec((NC, BN, 128), lambda i: (0, i, 0)),
                  pl.BlockSpec((NC, BN, 16), lambda i: (0, i, 0)),
                  pl.BlockSpec((H, HU), lambda i: (0, 0))],
        out_specs=pl.BlockSpec((BN, HU), lambda i: (i, 0)),
        out_shape=jax.ShapeDtypeStruct((N, HU), jnp.float32),
    )(ft_parts, dn_parts, m4)

    return out.reshape(N, H, U)
